# baseline (speedup 1.0000x reference)
import functools

import jax
import jax.numpy as jnp
from jax import lax
from jax.experimental import pallas as pl
from jax.experimental.pallas import tpu as pltpu
from jax.experimental.pallas import tpu_sc as plsc

NC = 2
NS = 16
NW = NC * NS
LANES = 16

CHUNK = 128
ITERS = 80
NPAD = 10240


def _sc_aggregate(x, src, dst):
    n, d = x.shape
    rpt = NPAD // NS
    epw = ITERS * CHUNK
    assert src.shape == (NW * epw,)

    mesh = plsc.VectorSubcoreMesh(
        core_axis_name="c", subcore_axis_name="s", num_cores=NC, num_subcores=NS
    )

    @functools.partial(
        pl.kernel,
        out_type=(
            jax.ShapeDtypeStruct((NC, NPAD, d), jnp.float32),
            jax.ShapeDtypeStruct((NC, NS, NPAD), jnp.float32),
        ),
        mesh=mesh,
        compiler_params=pltpu.CompilerParams(needs_layout_passes=False),
        scratch_types=(
            pltpu.VMEM((CHUNK,), jnp.int32),
            pltpu.VMEM((CHUNK,), jnp.int32),
            pltpu.VMEM((CHUNK, d), jnp.float32),
            pltpu.VMEM((NPAD,), jnp.float32),
            pltpu.VMEM_SHARED((NPAD, d), jnp.float32),
            pltpu.SemaphoreType.DMA,
        ),
    )
    def agg(x_hbm, src_hbm, dst_hbm, psum_hbm, pdeg_hbm,
            src_v, dst_v, rows_v, deg_v, acc, sem):
        cid = lax.axis_index("c")
        sid = lax.axis_index("s")
        wid = sid * NC + cid
        zvec = jnp.zeros((LANES,), jnp.float32)
        ones = jnp.ones((LANES,), jnp.float32)

        def fill_deg(i, carry):
            deg_v[pl.ds(i * LANES, LANES)] = zvec
            return carry
        lax.fori_loop(0, NPAD // LANES, fill_deg, 0)

        def fill_rows(i, carry):
            for j in range(d // LANES):
                rows_v[i, pl.ds(j * LANES, LANES)] = zvec
            return carry
        lax.fori_loop(0, CHUNK, fill_rows, 0)

        for c in range(rpt // CHUNK):
            pltpu.sync_copy(rows_v, acc.at[pl.ds(sid * rpt + c * CHUNK, CHUNK)])
        plsc.subcore_barrier()

        ebase = wid * epw

        def edge_step(i, carry):
            off = ebase + i * CHUNK
            pltpu.sync_copy(src_hbm.at[pl.ds(off, CHUNK)], src_v)
            pltpu.sync_copy(dst_hbm.at[pl.ds(off, CHUNK)], dst_v)
            pltpu.async_copy(x_hbm.at[src_v], rows_v, sem).wait()
            pltpu.sync_copy(rows_v, acc.at[dst_v], add=True)
            for k in range(CHUNK // LANES):
                dvec = dst_v[pl.ds(k * LANES, LANES)]
                plsc.addupdate_scatter(deg_v, [dvec], ones)
            return carry
        lax.fori_loop(0, ITERS, edge_step, 0)

        plsc.subcore_barrier()

        pltpu.sync_copy(deg_v, pdeg_hbm.at[cid, sid])
        pltpu.sync_copy(acc.at[pl.ds(sid * rpt, rpt)],
                        psum_hbm.at[cid, pl.ds(sid * rpt, rpt)])

    return agg(x, src, dst)


def _tc_finish(psum, pdeg, W, b2d):
    _, npad, d = psum.shape
    dout = W.shape[0]
    rblk = 1024
    grid = (npad // rblk,)

    def body(ps_ref, pd_ref, w_ref, b_ref, o_ref):
        s = ps_ref[0] + ps_ref[1]
        deg = jnp.sum(pd_ref[...], axis=(0, 1)).reshape(rblk, 1)
        mean = s / (deg + 1e-6)
        o_ref[...] = lax.dot_general(
            mean, w_ref[...], (((1,), (1,)), ((), ())),
            preferred_element_type=jnp.float32) + b_ref[...]

    return pl.pallas_call(
        body,
        grid=grid,
        in_specs=[
            pl.BlockSpec((NC, rblk, d), lambda i: (0, i, 0)),
            pl.BlockSpec((NC, NS, rblk), lambda i: (0, 0, i)),
            pl.BlockSpec((dout, d), lambda i: (0, 0)),
            pl.BlockSpec((1, dout), lambda i: (0, 0)),
        ],
        out_specs=pl.BlockSpec((rblk, dout), lambda i: (i, 0)),
        out_shape=jax.ShapeDtypeStruct((npad, dout), jnp.float32),
    )(psum, pdeg, W, b2d)


@jax.jit
def kernel(x, edge_index, W, b):
    n = x.shape[0]
    e = edge_index.shape[1]
    epad = NW * ITERS * CHUNK
    dst = edge_index[0].astype(jnp.int32)
    src = edge_index[1].astype(jnp.int32)
    src_p = jnp.concatenate([src, jnp.zeros((epad - e,), jnp.int32)])
    dst_p = jnp.concatenate([dst, jnp.full((epad - e,), n, jnp.int32)])
    psum, pdeg = _sc_aggregate(x, src_p, dst_p)
    out = _tc_finish(psum, pdeg, W, b.reshape(1, -1))
    return out[:n]


# speedup vs baseline: 2.0909x; 2.0909x over previous
import functools

import jax
import jax.numpy as jnp
from jax import lax
from jax.experimental import pallas as pl
from jax.experimental.pallas import tpu as pltpu
from jax.experimental.pallas import tpu_sc as plsc

NC = 2
NS = 16
NW = NC * NS
LANES = 16

CHUNK = 80
ITERS = 126
NPAD = 10240


def _sc_aggregate(x, src, dst):
    n, d = x.shape
    rpt = NPAD // NS
    epw = ITERS * CHUNK
    assert src.shape == (NW * epw,)

    mesh = plsc.VectorSubcoreMesh(
        core_axis_name="c", subcore_axis_name="s", num_cores=NC, num_subcores=NS
    )

    @functools.partial(
        pl.kernel,
        out_type=(
            jax.ShapeDtypeStruct((NC, NPAD, d), jnp.float32),
            jax.ShapeDtypeStruct((NC, NS, NPAD), jnp.float32),
        ),
        mesh=mesh,
        compiler_params=pltpu.CompilerParams(needs_layout_passes=False),
        scratch_types=(
            pltpu.VMEM((CHUNK,), jnp.int32),
            pltpu.VMEM((CHUNK,), jnp.int32),
            pltpu.VMEM((CHUNK,), jnp.int32),
            pltpu.VMEM((CHUNK,), jnp.int32),
            pltpu.VMEM((CHUNK, d), jnp.float32),
            pltpu.VMEM((CHUNK, d), jnp.float32),
            pltpu.VMEM((NPAD,), jnp.float32),
            pltpu.VMEM_SHARED((NPAD, d), jnp.float32),
            pltpu.SemaphoreType.DMA,
            pltpu.SemaphoreType.DMA,
        ),
    )
    def agg(x_hbm, src_hbm, dst_hbm, psum_hbm, pdeg_hbm,
            src_a, dst_a, src_b, dst_b, buf_a, buf_b, deg_v, acc,
            sem_a, sem_b):
        cid = lax.axis_index("c")
        sid = lax.axis_index("s")
        wid = sid * NC + cid
        zvec = jnp.zeros((LANES,), jnp.float32)
        ones = jnp.ones((LANES,), jnp.float32)

        def fill_deg(i, carry):
            deg_v[pl.ds(i * LANES, LANES)] = zvec
            return carry
        lax.fori_loop(0, NPAD // LANES, fill_deg, 0)

        def fill_rows(i, carry):
            for j in range(d // LANES):
                buf_a[i, pl.ds(j * LANES, LANES)] = zvec
            return carry
        lax.fori_loop(0, CHUNK, fill_rows, 0)

        for c in range(rpt // CHUNK):
            pltpu.sync_copy(buf_a, acc.at[pl.ds(sid * rpt + c * CHUNK, CHUNK)])
        plsc.subcore_barrier()

        ebase = wid * epw

        def load_idx_and_gather(i, sv, dv, buf, sem):
            off = ebase + i * CHUNK
            pltpu.sync_copy(src_hbm.at[pl.ds(off, CHUNK)], sv)
            pltpu.sync_copy(dst_hbm.at[pl.ds(off, CHUNK)], dv)
            pltpu.async_copy(x_hbm.at[sv], buf, sem)

        def consume(dv, buf, sem):
            pltpu.make_async_copy(x_hbm, buf, sem).wait()
            pltpu.sync_copy(buf, acc.at[dv], add=True)
            for k in range(CHUNK // LANES):
                dvec = dv[pl.ds(k * LANES, LANES)]
                plsc.addupdate_scatter(deg_v, [dvec], ones)

        load_idx_and_gather(0, src_a, dst_a, buf_a, sem_a)
        load_idx_and_gather(1, src_b, dst_b, buf_b, sem_b)

        def edge_step(t, carry):
            consume(dst_a, buf_a, sem_a)

            @pl.when(t + 1 < ITERS // 2)
            def _pf_a():
                load_idx_and_gather(2 * t + 2, src_a, dst_a, buf_a, sem_a)
            consume(dst_b, buf_b, sem_b)

            @pl.when(t + 1 < ITERS // 2)
            def _pf_b():
                load_idx_and_gather(2 * t + 3, src_b, dst_b, buf_b, sem_b)
            return carry
        lax.fori_loop(0, ITERS // 2, edge_step, 0)

        plsc.subcore_barrier()

        pltpu.sync_copy(deg_v, pdeg_hbm.at[cid, sid])
        pltpu.sync_copy(acc.at[pl.ds(sid * rpt, rpt)],
                        psum_hbm.at[cid, pl.ds(sid * rpt, rpt)])

    return agg(x, src, dst)


def _tc_finish(psum, pdeg, W, b2d):
    _, npad, d = psum.shape
    dout = W.shape[0]
    rblk = 1024
    grid = (npad // rblk,)

    def body(ps_ref, pd_ref, w_ref, b_ref, o_ref):
        s = ps_ref[0] + ps_ref[1]
        deg = jnp.sum(pd_ref[...], axis=(0, 1)).reshape(rblk, 1)
        mean = s / (deg + 1e-6)
        o_ref[...] = lax.dot_general(
            mean, w_ref[...], (((1,), (1,)), ((), ())),
            preferred_element_type=jnp.float32) + b_ref[...]

    return pl.pallas_call(
        body,
        grid=grid,
        in_specs=[
            pl.BlockSpec((NC, rblk, d), lambda i: (0, i, 0)),
            pl.BlockSpec((NC, NS, rblk), lambda i: (0, 0, i)),
            pl.BlockSpec((dout, d), lambda i: (0, 0)),
            pl.BlockSpec((1, dout), lambda i: (0, 0)),
        ],
        out_specs=pl.BlockSpec((rblk, dout), lambda i: (i, 0)),
        out_shape=jax.ShapeDtypeStruct((npad, dout), jnp.float32),
    )(psum, pdeg, W, b2d)


@jax.jit
def kernel(x, edge_index, W, b):
    n = x.shape[0]
    e = edge_index.shape[1]
    epad = NW * ITERS * CHUNK
    dst = edge_index[0].astype(jnp.int32)
    src = edge_index[1].astype(jnp.int32)
    src_p = jnp.concatenate([src, jnp.zeros((epad - e,), jnp.int32)])
    dst_p = jnp.concatenate([dst, jnp.full((epad - e,), n, jnp.int32)])
    psum, pdeg = _sc_aggregate(x, src_p, dst_p)
    out = _tc_finish(psum, pdeg, W, b.reshape(1, -1))
    return out[:n]
